# baseline (speedup 1.0000x reference)
import functools

import jax
import jax.numpy as jnp
from jax import lax
from jax.experimental import pallas as pl
from jax.experimental.pallas import tpu as pltpu


def _elu(z):
    return jnp.where(z > 0, z, jnp.exp(z) - 1.0)


def _body(x_ref, a0_ref, w1_ref, b1_ref, g1_ref, be1_ref, w2_ref,
          b2_ref, g2_ref, be2_ref, eps_ref, out_ref, z1_ref, s1_ref, s2_ref,
          *, rows, tiles):
    i = pl.program_id(0)
    xf = x_ref[...]
    half = rows // 2
    dot = functools.partial(
        lax.dot_general,
        dimension_numbers=(((1,), (0,)), ((), ())),
        preferred_element_type=jnp.float32,
        precision=lax.Precision.DEFAULT,
    )
    eps = eps_ref[0, 0]
    v = dot(a0_ref[...], xf)
    v = v + eps * x_ref[pl.ds(i * rows, rows), :]
    z1 = lax.dot_general(
        v, w1_ref[...], (((1,), (1,)), ((), ())),
        preferred_element_type=jnp.float32,
        precision=lax.Precision.HIGHEST,
    ) + b1_ref[...]
    z1_ref[pl.ds(i * rows, rows), :] = z1
    s1_ref[...] += jnp.sum(z1, axis=0, keepdims=True)
    s2_ref[...] += jnp.sum(z1 * z1, axis=0, keepdims=True)

    @pl.when(i == tiles - 1)
    def _finish():
        n = float(rows * tiles)
        z = z1_ref[...]
        mu1 = s1_ref[...] / n
        var1 = s2_ref[...] / n - mu1 * mu1
        h = g1_ref[...] * (z - mu1) * lax.rsqrt(var1 + 1e-5) + be1_ref[...]
        h = _elu(h)
        z2 = lax.dot_general(
            h, w2_ref[...], (((1,), (1,)), ((), ())),
            preferred_element_type=jnp.float32,
            precision=lax.Precision.HIGHEST,
        ) + b2_ref[...]
        mu2 = jnp.mean(z2, axis=0, keepdims=True)
        var2 = jnp.mean((z2 - mu2) ** 2, axis=0, keepdims=True)
        h2 = g2_ref[...] * (z2 - mu2) * lax.rsqrt(var2 + 1e-5) + be2_ref[...]
        out_ref[...] = _elu(h2)


def _init_scratch(s1_ref, s2_ref):
    s1_ref[...] = jnp.zeros_like(s1_ref)
    s2_ref[...] = jnp.zeros_like(s2_ref)


def kernel(x, a, W1, b1, gamma1, beta1, W2, b2, gamma2, beta2, epsilon):
    N, D = x.shape
    H = W1.shape[0]
    O = W2.shape[0]
    rows = 256
    tiles = N // rows

    full = lambda i: (0, 0)
    body = functools.partial(_body, rows=rows, tiles=tiles)

    def wrapped(*refs):
        i = pl.program_id(0)

        @pl.when(i == 0)
        def _():
            _init_scratch(refs[-2], refs[-1])

        body(*refs)

    return pl.pallas_call(
        wrapped,
        grid=(tiles,),
        in_specs=[
            pl.BlockSpec((N, D), full),
            pl.BlockSpec((rows, N), lambda i: (i, 0)),
            pl.BlockSpec((H, D), full),
            pl.BlockSpec((1, H), full),
            pl.BlockSpec((1, H), full),
            pl.BlockSpec((1, H), full),
            pl.BlockSpec((O, H), full),
            pl.BlockSpec((1, O), full),
            pl.BlockSpec((1, O), full),
            pl.BlockSpec((1, O), full),
            pl.BlockSpec((1, 1), full),
        ],
        out_specs=pl.BlockSpec((N, O), full),
        out_shape=jax.ShapeDtypeStruct((N, O), jnp.float32),
        scratch_shapes=[
            pltpu.VMEM((N, H), jnp.float32),
            pltpu.VMEM((1, H), jnp.float32),
            pltpu.VMEM((1, H), jnp.float32),
        ],
    )(x, a, W1, b1.reshape(1, H), gamma1.reshape(1, H), beta1.reshape(1, H),
      W2, b2.reshape(1, O), gamma2.reshape(1, O), beta2.reshape(1, O),
      epsilon)


# speedup vs baseline: 1.0327x; 1.0327x over previous
import functools

import jax
import jax.numpy as jnp
from jax import lax
from jax.experimental import pallas as pl
from jax.experimental.pallas import tpu as pltpu

_CHUNKS = (128, 128, 256) + (512,) * 7
_NBUF = 4
_BUF_ROWS = 512


def _elu(z):
    return jnp.where(z > 0, z, jnp.exp(z) - 1.0)


def _body(x_ref, a_hbm, w1_ref, b1_ref, g1_ref, be1_ref, w2_ref,
          b2_ref, g2_ref, be2_ref, eps_ref, out_ref,
          b0, b1s, b2s, b3, z1_ref, s0, s1, s2, s3, *, n_rows):
    bufs = (b0, b1s, b2s, b3)
    sems = (s0, s1, s2, s3)
    offs = [0]
    for r in _CHUNKS:
        offs.append(offs[-1] + r)

    def copy(c):
        b = c % _NBUF
        return pltpu.make_async_copy(
            a_hbm.at[pl.ds(offs[c], _CHUNKS[c]), :],
            bufs[b].at[pl.ds(0, _CHUNKS[c]), :],
            sems[b],
        )

    for c in range(_NBUF):
        copy(c).start()

    xf = x_ref[...]
    eps = eps_ref[0, 0]
    acc1 = jnp.zeros((1, z1_ref.shape[1]), jnp.float32)
    acc2 = jnp.zeros((1, z1_ref.shape[1]), jnp.float32)
    for c, rt in enumerate(_CHUNKS):
        b = c % _NBUF
        copy(c).wait()
        v = lax.dot_general(
            bufs[b][pl.ds(0, rt), :], xf, (((1,), (0,)), ((), ())),
            preferred_element_type=jnp.float32,
            precision=lax.Precision.DEFAULT,
        )
        v = v + eps * x_ref[pl.ds(offs[c], rt), :]
        z1 = lax.dot_general(
            v, w1_ref[...], (((1,), (1,)), ((), ())),
            preferred_element_type=jnp.float32,
            precision=lax.Precision.HIGHEST,
        ) + b1_ref[...]
        z1_ref[pl.ds(offs[c], rt), :] = z1
        acc1 = acc1 + jnp.sum(z1, axis=0, keepdims=True)
        acc2 = acc2 + jnp.sum(z1 * z1, axis=0, keepdims=True)
        if c + _NBUF < len(_CHUNKS):
            copy(c + _NBUF).start()

    n = float(n_rows)
    z = z1_ref[...]
    mu1 = acc1 / n
    var1 = acc2 / n - mu1 * mu1
    h = g1_ref[...] * (z - mu1) * lax.rsqrt(var1 + 1e-5) + be1_ref[...]
    h = _elu(h)
    z2 = lax.dot_general(
        h, w2_ref[...], (((1,), (1,)), ((), ())),
        preferred_element_type=jnp.float32,
        precision=lax.Precision.HIGHEST,
    ) + b2_ref[...]
    mu2 = jnp.mean(z2, axis=0, keepdims=True)
    var2 = jnp.mean((z2 - mu2) ** 2, axis=0, keepdims=True)
    h2 = g2_ref[...] * (z2 - mu2) * lax.rsqrt(var2 + 1e-5) + be2_ref[...]
    out_ref[...] = _elu(h2)


def kernel(x, a, W1, b1, gamma1, beta1, W2, b2, gamma2, beta2, epsilon):
    N, D = x.shape
    H = W1.shape[0]
    O = W2.shape[0]

    body = functools.partial(_body, n_rows=N)
    vspec = pl.BlockSpec(memory_space=pltpu.VMEM)
    return pl.pallas_call(
        body,
        in_specs=[
            vspec,
            pl.BlockSpec(memory_space=pl.ANY),
            vspec, vspec, vspec, vspec,
            vspec, vspec, vspec, vspec,
            vspec,
        ],
        out_specs=pl.BlockSpec(memory_space=pltpu.VMEM),
        out_shape=jax.ShapeDtypeStruct((N, O), jnp.float32),
        scratch_shapes=[pltpu.VMEM((_BUF_ROWS, N), jnp.float32)] * _NBUF
        + [pltpu.VMEM((N, H), jnp.float32)]
        + [pltpu.SemaphoreType.DMA] * _NBUF,
    )(x, a, W1, b1.reshape(1, H), gamma1.reshape(1, H), beta1.reshape(1, H),
      W2, b2.reshape(1, O), gamma2.reshape(1, O), beta2.reshape(1, O),
      epsilon)
